# baseline (speedup 1.0000x reference)
import functools

import jax
import jax.numpy as jnp
from jax import lax
from jax.experimental import pallas as pl
from jax.experimental.pallas import tpu as pltpu
from jax.experimental.pallas import tpu_sc as plsc

VOCAB1 = 1000001
BATCH = 16384
EMBED_DIM = 16
NUM_WORKERS = 32
B_PER_W = BATCH // NUM_WORKERS
CHUNK = 128
NCHUNK = B_PER_W // CHUNK
GROUPS = B_PER_W // 16

TBLK = 2048


def _transpose_body(x_ref, o_ref):
    o_ref[...] = x_ref[...].T


_transpose_table = pl.pallas_call(
    _transpose_body,
    grid=(pl.cdiv(VOCAB1, TBLK),),
    in_specs=[pl.BlockSpec((EMBED_DIM, TBLK), lambda i: (0, i))],
    out_specs=pl.BlockSpec((TBLK, EMBED_DIM), lambda i: (i, 0)),
    out_shape=jax.ShapeDtypeStruct((VOCAB1, EMBED_DIM), jnp.float32),
)

_mesh = plsc.VectorSubcoreMesh(core_axis_name="c", subcore_axis_name="s")


@functools.partial(
    pl.kernel,
    mesh=_mesh,
    out_type=jax.ShapeDtypeStruct((BATCH,), jnp.float32),
    scratch_types=[
        pltpu.VMEM((B_PER_W, 2), jnp.int32),
        pltpu.VMEM((NCHUNK, CHUNK), jnp.int32),
        pltpu.VMEM((NCHUNK, CHUNK), jnp.int32),
        pltpu.VMEM((B_PER_W, EMBED_DIM), jnp.float32),
        pltpu.VMEM((B_PER_W, EMBED_DIM), jnp.float32),
        pltpu.VMEM((B_PER_W,), jnp.float32),
        pltpu.VMEM((1,), jnp.float32),
        pltpu.SemaphoreType.DMA,
        pltpu.SemaphoreType.DMA,
    ],
    compiler_params=pltpu.CompilerParams(
        needs_layout_passes=False, use_tc_tiling_on_sc=False),
)
def _mf_kernel(pairs_hbm, utab_hbm, itab_hbm, bias_hbm, out_hbm,
               pairs_v, uidx_v, iidx_v, urows_v, irows_v, out_v, bias_v,
               sem_u, sem_i):
    wid = lax.axis_index("s") * 2 + lax.axis_index("c")
    base = wid * B_PER_W

    pltpu.sync_copy(pairs_hbm.at[pl.ds(base, B_PER_W)], pairs_v)
    pltpu.sync_copy(bias_hbm, bias_v)

    iota = lax.iota(jnp.int32, 16)
    zeros16 = jnp.zeros((16,), jnp.int32)
    ones16 = jnp.ones((16,), jnp.int32)

    for g in range(GROUPS):
        rows = g * 16 + iota
        c, off = divmod(g * 16, CHUNK)
        uidx_v[c, pl.ds(off, 16)] = plsc.load_gather(pairs_v, [rows, zeros16])
        iidx_v[c, pl.ds(off, 16)] = plsc.load_gather(pairs_v, [rows, ones16])

    copies = []
    for c in range(NCHUNK):
        dst = pl.ds(c * CHUNK, CHUNK)
        copies.append(pltpu.make_async_copy(
            utab_hbm.at[uidx_v.at[c]], urows_v.at[dst], sem_u))
        copies.append(pltpu.make_async_copy(
            itab_hbm.at[iidx_v.at[c]], irows_v.at[dst], sem_i))
    for cp in copies:
        cp.start()
    for cp in copies:
        cp.wait()

    bias_vec = plsc.load_gather(bias_v, [zeros16])

    def dot_body(g, carry):
        rows = g * 16 + iota
        acc = bias_vec
        for d in range(EMBED_DIM):
            cols = jnp.full((16,), d, jnp.int32)
            u = plsc.load_gather(urows_v, [rows, cols])
            v = plsc.load_gather(irows_v, [rows, cols])
            acc = acc + u * v
        out_v[pl.ds(g * 16, 16)] = acc
        return carry

    lax.fori_loop(0, GROUPS, dot_body, 0)

    pltpu.sync_copy(out_v, out_hbm.at[pl.ds(base, B_PER_W)])


def kernel(sparse_inputs, user_table, item_table, bias):
    pairs = sparse_inputs.astype(jnp.int32)
    ut_lin = _transpose_table(user_table.T)
    it_lin = _transpose_table(item_table.T)
    return _mf_kernel(pairs, ut_lin, it_lin, bias)


# speedup vs baseline: 1.3386x; 1.3386x over previous
import functools

import jax
import jax.numpy as jnp
from jax import lax
from jax.experimental import pallas as pl
from jax.experimental.pallas import tpu as pltpu
from jax.experimental.pallas import tpu_sc as plsc

VOCAB1 = 1000001
BATCH = 16384
EMBED_DIM = 16
NUM_WORKERS = 32
B_PER_W = BATCH // NUM_WORKERS
CHUNK = 128
NCHUNK = B_PER_W // CHUNK
GROUPS = B_PER_W // 16

TBLK = 8192


def _transpose_body(x_ref, o_ref):
    eye = jnp.eye(EMBED_DIM, dtype=jnp.float32)
    o_ref[...] = jax.lax.dot_general(
        x_ref[...], eye, (((0,), (0,)), ((), ())),
        preferred_element_type=jnp.float32)


_transpose_table = pl.pallas_call(
    _transpose_body,
    grid=(pl.cdiv(VOCAB1, TBLK),),
    in_specs=[pl.BlockSpec((EMBED_DIM, TBLK), lambda i: (0, i))],
    out_specs=pl.BlockSpec((TBLK, EMBED_DIM), lambda i: (i, 0)),
    out_shape=jax.ShapeDtypeStruct((VOCAB1, EMBED_DIM), jnp.float32),
)

_mesh = plsc.VectorSubcoreMesh(core_axis_name="c", subcore_axis_name="s")


@functools.partial(
    pl.kernel,
    mesh=_mesh,
    out_type=jax.ShapeDtypeStruct((BATCH,), jnp.float32),
    scratch_types=[
        pltpu.VMEM((B_PER_W, 2), jnp.int32),
        pltpu.VMEM((NCHUNK, CHUNK), jnp.int32),
        pltpu.VMEM((NCHUNK, CHUNK), jnp.int32),
        pltpu.VMEM((B_PER_W, EMBED_DIM), jnp.float32),
        pltpu.VMEM((B_PER_W, EMBED_DIM), jnp.float32),
        pltpu.VMEM((B_PER_W,), jnp.float32),
        pltpu.VMEM((1,), jnp.float32),
        pltpu.SemaphoreType.DMA,
        pltpu.SemaphoreType.DMA,
    ],
    compiler_params=pltpu.CompilerParams(
        needs_layout_passes=False, use_tc_tiling_on_sc=False),
)
def _mf_kernel(pairs_hbm, utab_hbm, itab_hbm, bias_hbm, out_hbm,
               pairs_v, uidx_v, iidx_v, urows_v, irows_v, out_v, bias_v,
               sem_u, sem_i):
    wid = lax.axis_index("s") * 2 + lax.axis_index("c")
    base = wid * B_PER_W

    pltpu.sync_copy(pairs_hbm.at[pl.ds(base, B_PER_W)], pairs_v)
    pltpu.sync_copy(bias_hbm, bias_v)

    iota = lax.iota(jnp.int32, 16)
    zeros16 = jnp.zeros((16,), jnp.int32)
    ones16 = jnp.ones((16,), jnp.int32)

    for g in range(GROUPS):
        rows = g * 16 + iota
        c, off = divmod(g * 16, CHUNK)
        uidx_v[c, pl.ds(off, 16)] = plsc.load_gather(pairs_v, [rows, zeros16])
        iidx_v[c, pl.ds(off, 16)] = plsc.load_gather(pairs_v, [rows, ones16])

    copies = []
    for c in range(NCHUNK):
        dst = pl.ds(c * CHUNK, CHUNK)
        copies.append(pltpu.make_async_copy(
            utab_hbm.at[uidx_v.at[c]], urows_v.at[dst], sem_u))
        copies.append(pltpu.make_async_copy(
            itab_hbm.at[iidx_v.at[c]], irows_v.at[dst], sem_i))
    for cp in copies:
        cp.start()
    for cp in copies:
        cp.wait()

    bias_vec = plsc.load_gather(bias_v, [zeros16])

    def dot_body(g, carry):
        rows = g * 16 + iota
        acc = bias_vec
        for d in range(EMBED_DIM):
            cols = jnp.full((16,), d, jnp.int32)
            u = plsc.load_gather(urows_v, [rows, cols])
            v = plsc.load_gather(irows_v, [rows, cols])
            acc = acc + u * v
        out_v[pl.ds(g * 16, 16)] = acc
        return carry

    lax.fori_loop(0, GROUPS, dot_body, 0)

    pltpu.sync_copy(out_v, out_hbm.at[pl.ds(base, B_PER_W)])


def kernel(sparse_inputs, user_table, item_table, bias):
    pairs = sparse_inputs.astype(jnp.int32)
    ut_lin = _transpose_table(user_table.T)
    it_lin = _transpose_table(item_table.T)
    return _mf_kernel(pairs, ut_lin, it_lin, bias)


# speedup vs baseline: 3.8021x; 2.8404x over previous
import functools

import jax
import jax.numpy as jnp
from jax import lax
from jax.experimental import pallas as pl
from jax.experimental.pallas import tpu as pltpu
from jax.experimental.pallas import tpu_sc as plsc

VOCAB1 = 1000001
BATCH = 16384
EMBED_DIM = 16
PACK = 8
NUM_WORKERS = 32
B_PER_W = BATCH // NUM_WORKERS
CHUNK = 128
NCHUNK = B_PER_W // CHUNK
GROUPS = B_PER_W // 16

TBLK = 8192
NBLK = (VOCAB1 + TBLK - 1) // TBLK
QROWS = TBLK // PACK
PACKED_ROWS = NBLK * QROWS


def _pack_body(x_ref, o_ref):
    eye = jnp.eye(EMBED_DIM, dtype=jnp.float32)
    acc = None
    for a in range(PACK):
        ea = jnp.pad(
            eye, ((0, 0), (a * EMBED_DIM, (PACK - 1 - a) * EMBED_DIM)))
        part = jax.lax.dot_general(
            x_ref[:, a * QROWS:(a + 1) * QROWS], ea, (((0,), (0,)), ((), ())),
            preferred_element_type=jnp.float32)
        acc = part if acc is None else acc + part
    o_ref[...] = acc


_pack_table = pl.pallas_call(
    _pack_body,
    grid=(pl.cdiv(VOCAB1, TBLK),),
    in_specs=[pl.BlockSpec((EMBED_DIM, TBLK), lambda i: (0, i))],
    out_specs=pl.BlockSpec((TBLK // PACK, PACK * EMBED_DIM),
                           lambda i: (i, 0)),
    out_shape=jax.ShapeDtypeStruct((PACKED_ROWS, PACK * EMBED_DIM),
                                   jnp.float32),
)

_mesh = plsc.VectorSubcoreMesh(core_axis_name="c", subcore_axis_name="s")


@functools.partial(
    pl.kernel,
    mesh=_mesh,
    out_type=jax.ShapeDtypeStruct((BATCH,), jnp.float32),
    scratch_types=[
        pltpu.VMEM((B_PER_W, 2), jnp.int32),
        pltpu.VMEM((NCHUNK, CHUNK), jnp.int32),
        pltpu.VMEM((B_PER_W, PACK * EMBED_DIM), jnp.float32),
        pltpu.VMEM((EMBED_DIM, B_PER_W), jnp.float32),
        pltpu.VMEM((B_PER_W,), jnp.float32),
        pltpu.VMEM((1,), jnp.float32),
        pltpu.SemaphoreType.DMA,
    ],
    compiler_params=pltpu.CompilerParams(
        needs_layout_passes=False, use_tc_tiling_on_sc=False),
)
def _mf_kernel(pairs_hbm, utab_hbm, itab_hbm, bias_hbm, out_hbm,
               pairs_v, q_v, rows_v, uemb_v, out_v, bias_v, sem):
    wid = lax.axis_index("s") * 2 + lax.axis_index("c")
    base = wid * B_PER_W

    pltpu.sync_copy(pairs_hbm.at[pl.ds(base, B_PER_W)], pairs_v)
    pltpu.sync_copy(bias_hbm, bias_v)

    iota = lax.iota(jnp.int32, 16)
    zeros16 = jnp.zeros((16,), jnp.int32)
    ones16 = jnp.ones((16,), jnp.int32)

    def _fire(col):
        sel = zeros16 if col == 0 else ones16
        for g in range(GROUPS):
            rows = g * 16 + iota
            c, off = divmod(g * 16, CHUNK)
            vals = plsc.load_gather(pairs_v, [rows, sel])
            q_v[c, pl.ds(off, 16)] = jnp.bitwise_or(
                lax.shift_left(lax.shift_right_logical(vals, 13), 10),
                jnp.bitwise_and(vals, QROWS - 1))
        copies = []
        for c in range(NCHUNK):
            copies.append(pltpu.make_async_copy(
                utab_hbm.at[q_v.at[c]] if col == 0
                else itab_hbm.at[q_v.at[c]],
                rows_v.at[pl.ds(c * CHUNK, CHUNK)], sem))
        for cp in copies:
            cp.start()
        for cp in copies:
            cp.wait()

    _fire(0)

    def extract_u(g, carry):
        rows = g * 16 + iota
        uvals = plsc.load_gather(pairs_v, [rows, zeros16])
        lb = lax.shift_left(
            jnp.bitwise_and(lax.shift_right_logical(uvals, 10), PACK - 1), 4)
        for d in range(EMBED_DIM):
            uemb_v[d, pl.ds(g * 16, 16)] = plsc.load_gather(
                rows_v, [rows, lb + d])
        return carry

    lax.fori_loop(0, GROUPS, extract_u, 0)

    _fire(1)

    bias_vec = plsc.load_gather(bias_v, [zeros16])

    def dot_body(g, carry):
        rows = g * 16 + iota
        sl = pl.ds(g * 16, 16)
        ivals = plsc.load_gather(pairs_v, [rows, ones16])
        lb = lax.shift_left(
            jnp.bitwise_and(lax.shift_right_logical(ivals, 10), PACK - 1), 4)
        acc = bias_vec
        for d in range(EMBED_DIM):
            iv = plsc.load_gather(rows_v, [rows, lb + d])
            acc = acc + uemb_v[d, sl] * iv
        out_v[sl] = acc
        return carry

    lax.fori_loop(0, GROUPS, dot_body, 0)

    pltpu.sync_copy(out_v, out_hbm.at[pl.ds(base, B_PER_W)])


def kernel(sparse_inputs, user_table, item_table, bias):
    pairs = sparse_inputs.astype(jnp.int32)
    ut_p = _pack_table(user_table.T)
    it_p = _pack_table(item_table.T)
    return _mf_kernel(pairs, ut_p, it_p, bias)
